# baseline (speedup 1.0000x reference)
import jax
import jax.numpy as jnp
from jax.experimental import pallas as pl
from jax.experimental.pallas import tpu as pltpu

_CHUNKS = 8


def _copy_body(u_ref, i_ref, ou_ref, oi_ref, sems):
    n_u = u_ref.shape[0]
    n_i = i_ref.shape[0]
    cu = n_u // _CHUNKS
    ci = n_i // _CHUNKS
    copies = []
    for c in range(_CHUNKS):
        lo_u = c * cu
        hi_u = n_u if c == _CHUNKS - 1 else (c + 1) * cu
        copies.append(pltpu.make_async_copy(
            u_ref.at[pl.ds(lo_u, hi_u - lo_u), :],
            ou_ref.at[pl.ds(lo_u, hi_u - lo_u), :],
            sems.at[2 * c]))
        lo_i = c * ci
        hi_i = n_i if c == _CHUNKS - 1 else (c + 1) * ci
        copies.append(pltpu.make_async_copy(
            i_ref.at[pl.ds(lo_i, hi_i - lo_i), :],
            oi_ref.at[pl.ds(lo_i, hi_i - lo_i), :],
            sems.at[2 * c + 1]))
    for cp in copies:
        cp.start()
    for cp in copies:
        cp.wait()


def kernel(user_emb, item_emb):
    n_u, d = user_emb.shape
    n_i, _ = item_emb.shape
    out_u, out_i = pl.pallas_call(
        _copy_body,
        in_specs=[
            pl.BlockSpec(memory_space=pl.ANY),
            pl.BlockSpec(memory_space=pl.ANY),
        ],
        out_specs=[
            pl.BlockSpec(memory_space=pl.ANY),
            pl.BlockSpec(memory_space=pl.ANY),
        ],
        out_shape=[
            jax.ShapeDtypeStruct((n_u, d), user_emb.dtype),
            jax.ShapeDtypeStruct((n_i, d), item_emb.dtype),
        ],
        scratch_shapes=[pltpu.SemaphoreType.DMA((2 * _CHUNKS,))],
    )(user_emb, item_emb)
    return (out_u, out_i)


# speedup vs baseline: 15.5588x; 15.5588x over previous
import jax
import jax.numpy as jnp
from jax.experimental import pallas as pl
from jax.experimental.pallas import tpu as pltpu

_CHUNK_ROWS = 25000


def _stream_table(src, dst, bufs, rsem, wsem, n_rows):
    nchunks = n_rows // _CHUNK_ROWS
    reads = []
    writes = []
    for k in range(nchunks):
        b = k % 2
        reads.append(pltpu.make_async_copy(
            src.at[pl.ds(k * _CHUNK_ROWS, _CHUNK_ROWS), :],
            bufs.at[b], rsem.at[b]))
        writes.append(pltpu.make_async_copy(
            bufs.at[b],
            dst.at[pl.ds(k * _CHUNK_ROWS, _CHUNK_ROWS), :],
            wsem.at[b]))
    reads[0].start()
    for k in range(nchunks):
        reads[k].wait()
        if k + 1 < nchunks:
            if k >= 1:
                writes[k - 1].wait()
            reads[k + 1].start()
        writes[k].start()
    writes[nchunks - 1].wait()
    if nchunks >= 2:
        writes[nchunks - 2].wait()


def _copy_body(u_ref, i_ref, ou_ref, oi_ref, bufs, rsem, wsem):
    _stream_table(u_ref, ou_ref, bufs, rsem, wsem, u_ref.shape[0])
    _stream_table(i_ref, oi_ref, bufs, rsem, wsem, i_ref.shape[0])


def kernel(user_emb, item_emb):
    n_u, d = user_emb.shape
    n_i, _ = item_emb.shape
    out_u, out_i = pl.pallas_call(
        _copy_body,
        in_specs=[
            pl.BlockSpec(memory_space=pl.ANY),
            pl.BlockSpec(memory_space=pl.ANY),
        ],
        out_specs=[
            pl.BlockSpec(memory_space=pl.ANY),
            pl.BlockSpec(memory_space=pl.ANY),
        ],
        out_shape=[
            jax.ShapeDtypeStruct((n_u, d), user_emb.dtype),
            jax.ShapeDtypeStruct((n_i, d), item_emb.dtype),
        ],
        scratch_shapes=[
            pltpu.VMEM((2, _CHUNK_ROWS, 64), jnp.float32),
            pltpu.SemaphoreType.DMA((2,)),
            pltpu.SemaphoreType.DMA((2,)),
        ],
    )(user_emb, item_emb)
    return (out_u, out_i)


# speedup vs baseline: 16.3013x; 1.0477x over previous
import jax
import jax.numpy as jnp
from jax.experimental import pallas as pl
from jax.experimental.pallas import tpu as pltpu

_CHUNK_ROWS = 12500
_NBUF = 8
_WRITE_DEPTH = 3


def _copy_body(u_ref, i_ref, ou_ref, oi_ref, bufs, rsem, wsem):
    chunks = []
    for (src, dst) in ((u_ref, ou_ref), (i_ref, oi_ref)):
        for k in range(src.shape[0] // _CHUNK_ROWS):
            chunks.append((src, dst, k * _CHUNK_ROWS))
    n = len(chunks)
    reads, writes = [], []
    for idx, (src, dst, off) in enumerate(chunks):
        b = idx % _NBUF
        reads.append(pltpu.make_async_copy(
            src.at[pl.ds(off, _CHUNK_ROWS), :], bufs.at[b], rsem.at[b]))
        writes.append(pltpu.make_async_copy(
            bufs.at[b], dst.at[pl.ds(off, _CHUNK_ROWS), :], wsem.at[b]))

    read_ahead = _NBUF - _WRITE_DEPTH
    waited = set()
    for j in range(min(read_ahead, n)):
        reads[j].start()
    for k in range(n):
        reads[k].wait()
        writes[k].start()
        j = k + read_ahead
        if j < n:
            jw = j - _NBUF
            if jw >= 0:
                writes[jw].wait()
                waited.add(jw)
            reads[j].start()
    for k in range(n):
        if k not in waited:
            writes[k].wait()


def kernel(user_emb, item_emb):
    n_u, d = user_emb.shape
    n_i, _ = item_emb.shape
    out_u, out_i = pl.pallas_call(
        _copy_body,
        in_specs=[
            pl.BlockSpec(memory_space=pl.ANY),
            pl.BlockSpec(memory_space=pl.ANY),
        ],
        out_specs=[
            pl.BlockSpec(memory_space=pl.ANY),
            pl.BlockSpec(memory_space=pl.ANY),
        ],
        out_shape=[
            jax.ShapeDtypeStruct((n_u, d), user_emb.dtype),
            jax.ShapeDtypeStruct((n_i, d), item_emb.dtype),
        ],
        scratch_shapes=[
            pltpu.VMEM((_NBUF, _CHUNK_ROWS, 64), jnp.float32),
            pltpu.SemaphoreType.DMA((_NBUF,)),
            pltpu.SemaphoreType.DMA((_NBUF,)),
        ],
    )(user_emb, item_emb)
    return (out_u, out_i)


# speedup vs baseline: 16.3353x; 1.0021x over previous
import jax
import jax.numpy as jnp
from jax.experimental import pallas as pl
from jax.experimental.pallas import tpu as pltpu

_BLOCK_ROWS = 8192


def _copy_body(u_ref, i_ref, ou_ref, oi_ref):
    ou_ref[...] = u_ref[...]
    oi_ref[...] = i_ref[...]


def kernel(user_emb, item_emb):
    n_u, d = user_emb.shape
    n_i, _ = item_emb.shape
    grid = (pl.cdiv(max(n_u, n_i), _BLOCK_ROWS),)
    out_u, out_i = pl.pallas_call(
        _copy_body,
        grid=grid,
        in_specs=[
            pl.BlockSpec((_BLOCK_ROWS, d), lambda r: (r, 0)),
            pl.BlockSpec((_BLOCK_ROWS, d), lambda r: (r, 0)),
        ],
        out_specs=[
            pl.BlockSpec((_BLOCK_ROWS, d), lambda r: (r, 0)),
            pl.BlockSpec((_BLOCK_ROWS, d), lambda r: (r, 0)),
        ],
        out_shape=[
            jax.ShapeDtypeStruct((n_u, d), user_emb.dtype),
            jax.ShapeDtypeStruct((n_i, d), item_emb.dtype),
        ],
        compiler_params=pltpu.CompilerParams(
            dimension_semantics=("parallel",),
        ),
    )(user_emb, item_emb)
    return (out_u, out_i)
